# baseline (speedup 1.0000x reference)
import functools

import jax
import jax.numpy as jnp
from jax import lax
from jax.experimental import pallas as pl
from jax.experimental.pallas import tpu as pltpu
from jax.experimental.pallas import tpu_sc as plsc

_N_SIZES = [10000, 20000, 40000]
_N_PADS = [10240, 20480, 40960]
_BN = 512
_CH = 128
_RANGE2 = 20480
_AR_EXTRA = 512


def _round_up(v, m):
    return (v + m - 1) // m * m


def _mm_body(x_ref, w_ref, o0_ref, o1_ref, *, chunks, w0w):
    y = jnp.dot(x_ref[...], w_ref[...], preferred_element_type=jnp.float32,
                precision=jax.lax.Precision.DEFAULT)
    o0_ref[...] = y[:, :w0w]
    for j in range(chunks):
        o1_ref[j] = y[:, w0w + 128 * j:w0w + 128 * (j + 1)]


def _matmul_tables(x, wcat, chunks, w0w):
    n_rows, d_pad = x.shape
    f = wcat.shape[1]
    grid = (n_rows // _BN,)
    xw0, xw1 = pl.pallas_call(
        functools.partial(_mm_body, chunks=chunks, w0w=w0w),
        grid=grid,
        in_specs=[
            pl.BlockSpec((_BN, d_pad), lambda i: (i, 0)),
            pl.BlockSpec((d_pad, f), lambda i: (0, 0)),
        ],
        out_specs=[
            pl.BlockSpec((_BN, w0w), lambda i: (i, 0)),
            pl.BlockSpec((chunks, _BN, 128), lambda i: (0, i, 0)),
        ],
        out_shape=[
            jax.ShapeDtypeStruct((n_rows, w0w), jnp.float32),
            jax.ShapeDtypeStruct((chunks, n_rows, 128), jnp.float32),
        ],
    )(x, wcat)
    return xw0, xw1.reshape(chunks * n_rows, 128)


def _pick_unroll(nch):
    return 4 if nch % 4 == 0 else (2 if nch % 2 == 0 else 1)


def _sc_seg(n_rows, ar, e_pad, chunks, ranges):
    ept = e_pad // 16
    nch = ept // _CH
    u = _pick_unroll(nch)
    ngroups = nch // u
    rpt = ar // 16
    total = chunks * ranges
    pps = (total + 1) // 2
    mesh = plsc.VectorSubcoreMesh(core_axis_name="c", subcore_axis_name="s")
    scratch = ([pltpu.VMEM((_CH,), jnp.int32) for _ in range(u)]
               + [pltpu.VMEM((_CH,), jnp.int32)]
               + [pltpu.VMEM((_CH, 128), jnp.float32) for _ in range(u)]
               + [pltpu.VMEM_SHARED((ar, 128), jnp.float32)]
               + [pltpu.SemaphoreType.DMA for _ in range(u)])

    def body(xw1_hbm, srcoff_hbm, dstr_hbm, zeros_hbm, out_hbm, *scr):
        srcis = scr[:u]
        dsti = scr[u]
        rows = scr[u + 1:2 * u + 1]
        acc = scr[2 * u + 1]
        sems = scr[2 * u + 2:]
        cc = lax.axis_index("c")
        ss = lax.axis_index("s")
        for p in range(pps):
            q = jnp.minimum(cc * pps + p, total - 1)
            j = q // ranges
            r = q % ranges
            pltpu.sync_copy(zeros_hbm.at[pl.ds(ss * rpt, rpt)],
                            acc.at[pl.ds(ss * rpt, rpt)])
            plsc.subcore_barrier()
            ebase = j * e_pad + ss * ept
            dbase = r * e_pad + ss * ept

            def group(g, carry):
                for uu in range(u):
                    k = g * u + uu
                    pltpu.sync_copy(
                        srcoff_hbm.at[pl.ds(ebase + k * _CH, _CH)], srcis[uu])
                    pltpu.async_copy(xw1_hbm.at[srcis[uu]], rows[uu], sems[uu])
                for uu in range(u):
                    k = g * u + uu
                    pltpu.make_async_copy(xw1_hbm.at[srcis[uu]], rows[uu],
                                          sems[uu]).wait()
                    pltpu.sync_copy(
                        dstr_hbm.at[pl.ds(dbase + k * _CH, _CH)], dsti)
                    pltpu.sync_copy(rows[uu], acc.at[dsti], add=True)
                return carry

            lax.fori_loop(0, ngroups, group, 0)
            plsc.subcore_barrier()
            obase = q * ar + ss * rpt
            pltpu.sync_copy(acc.at[pl.ds(ss * rpt, rpt)],
                            out_hbm.at[pl.ds(obase, rpt)])

    return pl.kernel(
        body,
        out_type=[jax.ShapeDtypeStruct((total * ar, 128), jnp.float32)],
        mesh=mesh, scratch_types=scratch)


def _ep_body(*refs, widths, act, with_res):
    o_ref = refs[-1]
    chunks = len(widths)
    xw0 = refs[0][...]
    parts = [refs[1 + j][0, 0][:, :widths[j]] for j in range(chunks)]
    t = jnp.concatenate(parts, axis=1) if chunks > 1 else parts[0]
    t = xw0 + t + refs[1 + chunks][...]
    if act:
        t = jnp.maximum(t, 0.0)
    if with_res:
        t = 0.5 * (refs[2 + chunks][...] + t)
    o_ref[...] = t


def _epilogue(xw0, aggf, widths, ranges, ar, range_size, n_rows, b, act, res):
    chunks = len(widths)
    agg = aggf.reshape(chunks, ranges, ar, 128)
    f = sum(widths)
    grid = (n_rows // _BN,)
    bpr = range_size // _BN

    def mk_map(j):
        return lambda i: (j, (i // bpr) % ranges, i % bpr, 0)

    in_specs = [pl.BlockSpec((_BN, f), lambda i: (i, 0))]
    in_specs += [pl.BlockSpec((1, 1, _BN, 128), mk_map(j))
                 for j in range(chunks)]
    in_specs.append(pl.BlockSpec((1, f), lambda i: (0, 0)))
    args = [xw0] + [agg] * chunks + [b]
    if res is not None:
        in_specs.append(pl.BlockSpec((_BN, f), lambda i: (i, 0)))
        args.append(res)
    return pl.pallas_call(
        functools.partial(_ep_body, widths=tuple(widths), act=act,
                          with_res=res is not None),
        grid=grid,
        in_specs=in_specs,
        out_specs=pl.BlockSpec((_BN, f), lambda i: (i, 0)),
        out_shape=jax.ShapeDtypeStruct((n_rows, f), jnp.float32),
    )(*args)


def _graph_conv(x_pad, p, ed, act=True, res=None):
    n_rows, d_pad = x_pad.shape
    e_pad = ed["e_pad"]
    ranges = ed["ranges"]
    range_size = ed["range_size"]
    ar = range_size + _AR_EXTRA
    dout = p["W0"].shape[1]
    if dout == 3:
        chunks, widths, w0w = 1, [16], 16
    elif dout == 96:
        chunks, widths, w0w = 1, [96], 96
    else:
        chunks, widths, w0w = 2, [128, 64], 192
    din = p["W0"].shape[0]
    w0 = jnp.pad(p["W0"], ((0, d_pad - din), (0, w0w - dout)))
    w1 = jnp.pad(p["W1"], ((0, d_pad - din), (0, 128 * chunks - dout)))
    wcat = jnp.concatenate([w0, w1], axis=1)
    b = jnp.pad(p["b"], (0, w0w - dout))[None, :]
    xw0, xw1 = _matmul_tables(x_pad, wcat, chunks, w0w)
    srcoff = ed["src"] if chunks == 1 else ed["srcoff2"]
    zeros = jnp.zeros((ar, 128), jnp.float32)
    (aggf,) = _sc_seg(n_rows, ar, e_pad, chunks, ranges)(
        xw1, srcoff, ed["dstr"], zeros)
    return _epilogue(xw0, aggf, widths, ranges, ar, range_size, n_rows,
                     b, act, res)


def _conv(x, w, b, stride=1):
    y = jax.lax.conv_general_dilated(x, w, (stride, stride), "SAME",
                                     dimension_numbers=("NHWC", "HWIO", "NHWC"))
    return jax.nn.relu(y + b)


def _cnn18(img, cnn):
    x = img[None]
    feats = []
    for i in range(6):
        p = cnn[i]
        x = _conv(x, p["c1W"], p["c1b"])
        x = _conv(x, p["c2W"], p["c2b"])
        if i >= 2:
            feats.append(x[0])
        x = _conv(x, p["sW"], p["sb"], 2)
    return feats


def _bilinear(feat, u, v):
    s = feat.shape[0]
    u0 = jnp.clip(jnp.floor(u).astype(jnp.int32), 0, s - 1)
    v0 = jnp.clip(jnp.floor(v).astype(jnp.int32), 0, s - 1)
    u1 = jnp.clip(u0 + 1, 0, s - 1)
    v1 = jnp.clip(v0 + 1, 0, s - 1)
    du = (u - u0.astype(u.dtype))[:, None]
    dv = (v - v0.astype(v.dtype))[:, None]
    f00 = feat[v0, u0]
    f01 = feat[v0, u1]
    f10 = feat[v1, u0]
    f11 = feat[v1, u1]
    return (f00 * (1 - du) * (1 - dv) + f01 * du * (1 - dv)
            + f10 * (1 - du) * dv + f11 * du * dv)


def _projection(x, img_feats):
    xc, yc = x[:, 0], x[:, 1]
    parts = [x]
    for feat in img_feats:
        s = feat.shape[0]
        u = (jnp.tanh(xc) * 0.5 + 0.5) * (s - 1)
        v = (jnp.tanh(yc) * 0.5 + 0.5) * (s - 1)
        parts.append(_bilinear(feat, u, v))
    return jnp.concatenate(parts, axis=1)


def _unpool(x, idx):
    new = 0.5 * (x[idx[:, 0]] + x[idx[:, 1]])
    return jnp.concatenate([x, new], axis=0)


def _prep_edges(ei, n, n_rows, ranges, range_size):
    e = ei.shape[1]
    e_pad = _round_up(e, 8192)
    src = jnp.concatenate([ei[0], jnp.zeros((e_pad - e,), jnp.int32)])
    dst = jnp.concatenate([ei[1], jnp.full((e_pad - e,), n, jnp.int32)])
    srcoff2 = jnp.concatenate([src, src + n_rows])
    drs = []
    for r in range(ranges):
        lo = r * range_size
        inr = (dst >= lo) & (dst < lo + range_size)
        drs.append(jnp.where(inr, dst - lo, range_size))
    return {"e_pad": e_pad, "src": src, "srcoff2": srcoff2,
            "dstr": jnp.concatenate(drs), "ranges": ranges,
            "range_size": range_size}


_STAGE_RANGES = [(2, 7168), (3, 7168), (6, 7168)]


def kernel(img_input, features, edge_index0, edge_index1, edge_index2,
           pool_idx0, pool_idx1, params):
    eis = [edge_index0, edge_index1, edge_index2]
    pis = [pool_idx0, pool_idx1]
    img_feats = _cnn18(img_input, params["cnn"])
    x = features
    outputs, outputs_unpool = [], []
    x_conv = None
    for i in range(3):
        n = _N_SIZES[i]
        n_rows = _N_PADS[i] + _AR_EXTRA
        ranges, range_size = _STAGE_RANGES[i]
        ed = _prep_edges(eis[i], n, n_rows, ranges, range_size)
        x_proj = _projection(x, img_feats)
        if i > 0:
            outputs_unpool.append(_unpool(x, pis[i - 1]))
            x_proj = jnp.concatenate([x_proj, x_conv], axis=1)
            x_proj = _unpool(x_proj, pis[i - 1])
        d = x_proj.shape[1]
        d_pad = _round_up(d, 128)
        xp = jnp.pad(x_proj, ((0, n_rows - n), (0, d_pad - d)))
        st = params["gcn"][i]
        h = _graph_conv(xp, st["gc_in"], ed, act=True)
        for rb in st["res"]:
            h1 = _graph_conv(h, rb["gc1"], ed, act=True)
            h = _graph_conv(h1, rb["gc2"], ed, act=True, res=h)
        x_conv = h[:n]
        if i == 2:
            y = _graph_conv(h, st["final"][0], ed, act=True)
            yp = jnp.pad(y, ((0, 0), (0, 128 - y.shape[1])))
            xo = _graph_conv(yp, st["final"][1], ed, act=False)
        else:
            xo = _graph_conv(h, st["final"][0], ed, act=False)
        x = xo[:n, :3]
        outputs.append(x)
    return tuple(outputs) + tuple(outputs_unpool)


# speedup vs baseline: 1.4855x; 1.4855x over previous
import functools

import jax
import jax.numpy as jnp
from jax import lax
from jax.experimental import pallas as pl
from jax.experimental.pallas import tpu as pltpu
from jax.experimental.pallas import tpu_sc as plsc

_N_SIZES = [10000, 20000, 40000]
_N_PADS = [10240, 20480, 40960]
_BN = 512
_CH = 128
_RANGE2 = 20480
_AR_EXTRA = 512


def _round_up(v, m):
    return (v + m - 1) // m * m


def _mm_body(x_ref, w_ref, o0_ref, o1_ref, *, chunks, w0w):
    y = jnp.dot(x_ref[...], w_ref[...], preferred_element_type=jnp.float32,
                precision=jax.lax.Precision.DEFAULT)
    o0_ref[...] = y[:, :w0w]
    for j in range(chunks):
        o1_ref[j] = y[:, w0w + 128 * j:w0w + 128 * (j + 1)]


def _matmul_tables(x, wcat, chunks, w0w):
    n_rows, d_pad = x.shape
    f = wcat.shape[1]
    grid = (n_rows // _BN,)
    xw0, xw1 = pl.pallas_call(
        functools.partial(_mm_body, chunks=chunks, w0w=w0w),
        grid=grid,
        in_specs=[
            pl.BlockSpec((_BN, d_pad), lambda i: (i, 0)),
            pl.BlockSpec((d_pad, f), lambda i: (0, 0)),
        ],
        out_specs=[
            pl.BlockSpec((_BN, w0w), lambda i: (i, 0)),
            pl.BlockSpec((chunks, _BN, 128), lambda i: (0, i, 0)),
        ],
        out_shape=[
            jax.ShapeDtypeStruct((n_rows, w0w), jnp.float32),
            jax.ShapeDtypeStruct((chunks, n_rows, 128), jnp.float32),
        ],
    )(x, wcat)
    return xw0, xw1.reshape(chunks * n_rows, 128)


def _pick_unroll(nch):
    return 4 if nch % 4 == 0 else (2 if nch % 2 == 0 else 1)


def _sc_seg(n_rows, ar, e_pad, chunks, ranges):
    ept = e_pad // 16
    nch = ept // _CH
    ngroups = nch // 2
    rpt = ar // 16
    erows = e_pad // _CH
    total = chunks * ranges
    pps = (total + 1) // 2
    mesh = plsc.VectorSubcoreMesh(core_axis_name="c", subcore_axis_name="s")
    scratch = ([pltpu.VMEM((2, _CH), jnp.int32) for _ in range(4)]
               + [pltpu.VMEM((_CH, 128), jnp.float32) for _ in range(2)]
               + [pltpu.VMEM_SHARED((ar, 128), jnp.float32)]
               + [pltpu.SemaphoreType.DMA for _ in range(2)])

    def body(xw1_hbm, srcoff_hbm, dstr_hbm, zeros_hbm, out_hbm, *scr):
        sb0, db0, sb1, db1 = scr[0:4]
        rows = scr[4:6]
        acc = scr[6]
        sems = scr[7:9]
        cc = lax.axis_index("c")
        ss = lax.axis_index("s")
        for p in range(pps):
            q = jnp.minimum(cc * pps + p, total - 1)
            j = q // ranges
            r = q % ranges
            pltpu.sync_copy(zeros_hbm, acc.at[pl.ds(ss * rpt, rpt)])
            plsc.subcore_barrier()
            sbase = j * erows + ss * nch
            dbase = r * erows + ss * nch
            pltpu.sync_copy(srcoff_hbm.at[pl.ds(sbase, 2)], sb0)
            pltpu.sync_copy(dstr_hbm.at[pl.ds(dbase, 2)], db0)
            pltpu.async_copy(xw1_hbm.at[sb0.at[0]], rows[0], sems[0])
            pltpu.async_copy(xw1_hbm.at[sb0.at[1]], rows[1], sems[1])

            def pair(go, carry):
                for par, sb, db, sbn, dbn in ((0, sb0, db0, sb1, db1),
                                              (1, sb1, db1, sb0, db0)):
                    g = 2 * go + par

                    @pl.when(g + 1 < ngroups)
                    def _():
                        pltpu.sync_copy(
                            srcoff_hbm.at[pl.ds(sbase + 2 * (g + 1), 2)], sbn)
                        pltpu.sync_copy(
                            dstr_hbm.at[pl.ds(dbase + 2 * (g + 1), 2)], dbn)
                    for uu in range(2):
                        k = 2 * g + uu
                        pltpu.make_async_copy(xw1_hbm.at[sb.at[uu]], rows[uu],
                                              sems[uu]).wait()
                        pltpu.sync_copy(rows[uu], acc.at[db.at[uu]], add=True)

                        @pl.when(k + 2 < nch)
                        def _():
                            pltpu.async_copy(xw1_hbm.at[sbn.at[uu]], rows[uu],
                                             sems[uu])
                return carry

            lax.fori_loop(0, ngroups // 2, pair, 0)
            plsc.subcore_barrier()
            obase = q * ar + ss * rpt
            pltpu.sync_copy(acc.at[pl.ds(ss * rpt, rpt)],
                            out_hbm.at[pl.ds(obase, rpt)])

    return pl.kernel(
        body,
        out_type=[jax.ShapeDtypeStruct((total * ar, 128), jnp.float32)],
        mesh=mesh, scratch_types=scratch)


def _ep_body(*refs, widths, act, with_res):
    o_ref = refs[-1]
    chunks = len(widths)
    xw0 = refs[0][...]
    parts = [refs[1 + j][0, 0][:, :widths[j]] for j in range(chunks)]
    t = jnp.concatenate(parts, axis=1) if chunks > 1 else parts[0]
    t = xw0 + t + refs[1 + chunks][...]
    if act:
        t = jnp.maximum(t, 0.0)
    if with_res:
        t = 0.5 * (refs[2 + chunks][...] + t)
    o_ref[...] = t


def _epilogue(xw0, aggf, widths, ranges, ar, range_size, n_rows, b, act, res):
    chunks = len(widths)
    agg = aggf.reshape(chunks, ranges, ar, 128)
    f = sum(widths)
    grid = (n_rows // _BN,)
    bpr = range_size // _BN

    def mk_map(j):
        return lambda i: (j, (i // bpr) % ranges, i % bpr, 0)

    in_specs = [pl.BlockSpec((_BN, f), lambda i: (i, 0))]
    in_specs += [pl.BlockSpec((1, 1, _BN, 128), mk_map(j))
                 for j in range(chunks)]
    in_specs.append(pl.BlockSpec((1, f), lambda i: (0, 0)))
    args = [xw0] + [agg] * chunks + [b]
    if res is not None:
        in_specs.append(pl.BlockSpec((_BN, f), lambda i: (i, 0)))
        args.append(res)
    return pl.pallas_call(
        functools.partial(_ep_body, widths=tuple(widths), act=act,
                          with_res=res is not None),
        grid=grid,
        in_specs=in_specs,
        out_specs=pl.BlockSpec((_BN, f), lambda i: (i, 0)),
        out_shape=jax.ShapeDtypeStruct((n_rows, f), jnp.float32),
    )(*args)


def _graph_conv(x_pad, p, ed, act=True, res=None):
    n_rows, d_pad = x_pad.shape
    e_pad = ed["e_pad"]
    ranges = ed["ranges"]
    range_size = ed["range_size"]
    ar = range_size + _AR_EXTRA
    dout = p["W0"].shape[1]
    if dout == 3:
        chunks, widths, w0w = 1, [16], 16
    elif dout == 96:
        chunks, widths, w0w = 1, [96], 96
    else:
        chunks, widths, w0w = 2, [128, 64], 192
    din = p["W0"].shape[0]
    w0 = jnp.pad(p["W0"], ((0, d_pad - din), (0, w0w - dout)))
    w1 = jnp.pad(p["W1"], ((0, d_pad - din), (0, 128 * chunks - dout)))
    wcat = jnp.concatenate([w0, w1], axis=1)
    b = jnp.pad(p["b"], (0, w0w - dout))[None, :]
    xw0, xw1 = _matmul_tables(x_pad, wcat, chunks, w0w)
    srcoff = ed["src"] if chunks == 1 else ed["srcoff2"]
    zeros = jnp.zeros((ar // 16, 128), jnp.float32)
    (aggf,) = _sc_seg(n_rows, ar, e_pad, chunks, ranges)(
        xw1, srcoff, ed["dstr"], zeros)
    return _epilogue(xw0, aggf, widths, ranges, ar, range_size, n_rows,
                     b, act, res)


def _conv(x, w, b, stride=1):
    y = jax.lax.conv_general_dilated(x, w, (stride, stride), "SAME",
                                     dimension_numbers=("NHWC", "HWIO", "NHWC"))
    return jax.nn.relu(y + b)


def _cnn18(img, cnn):
    x = img[None]
    feats = []
    for i in range(6):
        p = cnn[i]
        x = _conv(x, p["c1W"], p["c1b"])
        x = _conv(x, p["c2W"], p["c2b"])
        if i >= 2:
            feats.append(x[0])
        x = _conv(x, p["sW"], p["sb"], 2)
    return feats


def _bilinear(feat, u, v):
    s = feat.shape[0]
    u0 = jnp.clip(jnp.floor(u).astype(jnp.int32), 0, s - 1)
    v0 = jnp.clip(jnp.floor(v).astype(jnp.int32), 0, s - 1)
    u1 = jnp.clip(u0 + 1, 0, s - 1)
    v1 = jnp.clip(v0 + 1, 0, s - 1)
    du = (u - u0.astype(u.dtype))[:, None]
    dv = (v - v0.astype(v.dtype))[:, None]
    f00 = feat[v0, u0]
    f01 = feat[v0, u1]
    f10 = feat[v1, u0]
    f11 = feat[v1, u1]
    return (f00 * (1 - du) * (1 - dv) + f01 * du * (1 - dv)
            + f10 * (1 - du) * dv + f11 * du * dv)


def _projection(x, img_feats):
    xc, yc = x[:, 0], x[:, 1]
    parts = [x]
    for feat in img_feats:
        s = feat.shape[0]
        u = (jnp.tanh(xc) * 0.5 + 0.5) * (s - 1)
        v = (jnp.tanh(yc) * 0.5 + 0.5) * (s - 1)
        parts.append(_bilinear(feat, u, v))
    return jnp.concatenate(parts, axis=1)


def _unpool(x, idx):
    new = 0.5 * (x[idx[:, 0]] + x[idx[:, 1]])
    return jnp.concatenate([x, new], axis=0)


def _prep_edges(ei, n, n_rows, ranges, range_size):
    e = ei.shape[1]
    e_pad = _round_up(e, 8192)
    src = jnp.concatenate([ei[0], jnp.zeros((e_pad - e,), jnp.int32)])
    dst = jnp.concatenate([ei[1], jnp.full((e_pad - e,), n, jnp.int32)])
    srcoff2 = jnp.concatenate([src, src + n_rows])
    drs = []
    for r in range(ranges):
        lo = r * range_size
        inr = (dst >= lo) & (dst < lo + range_size)
        drs.append(jnp.where(inr, dst - lo, range_size))
    return {"e_pad": e_pad, "src": src.reshape(-1, _CH),
            "srcoff2": srcoff2.reshape(-1, _CH),
            "dstr": jnp.concatenate(drs).reshape(-1, _CH),
            "ranges": ranges, "range_size": range_size}


_STAGE_RANGES = [(1, 10240), (2, 10240), (4, 10240)]


def kernel(img_input, features, edge_index0, edge_index1, edge_index2,
           pool_idx0, pool_idx1, params):
    eis = [edge_index0, edge_index1, edge_index2]
    pis = [pool_idx0, pool_idx1]
    img_feats = _cnn18(img_input, params["cnn"])
    x = features
    outputs, outputs_unpool = [], []
    x_conv = None
    for i in range(3):
        n = _N_SIZES[i]
        n_rows = _N_PADS[i] + _AR_EXTRA
        ranges, range_size = _STAGE_RANGES[i]
        ed = _prep_edges(eis[i], n, n_rows, ranges, range_size)
        x_proj = _projection(x, img_feats)
        if i > 0:
            outputs_unpool.append(_unpool(x, pis[i - 1]))
            x_proj = jnp.concatenate([x_proj, x_conv], axis=1)
            x_proj = _unpool(x_proj, pis[i - 1])
        d = x_proj.shape[1]
        d_pad = _round_up(d, 128)
        xp = jnp.pad(x_proj, ((0, n_rows - n), (0, d_pad - d)))
        st = params["gcn"][i]
        h = _graph_conv(xp, st["gc_in"], ed, act=True)
        for rb in st["res"]:
            h1 = _graph_conv(h, rb["gc1"], ed, act=True)
            h = _graph_conv(h1, rb["gc2"], ed, act=True, res=h)
        x_conv = h[:n]
        if i == 2:
            y = _graph_conv(h, st["final"][0], ed, act=True)
            yp = jnp.pad(y, ((0, 0), (0, 128 - y.shape[1])))
            xo = _graph_conv(yp, st["final"][1], ed, act=False)
        else:
            xo = _graph_conv(h, st["final"][0], ed, act=False)
        x = xo[:n, :3]
        outputs.append(x)
    return tuple(outputs) + tuple(outputs_unpool)


# speedup vs baseline: 1.5667x; 1.0546x over previous
import functools

import jax
import jax.numpy as jnp
from jax import lax
from jax.experimental import pallas as pl
from jax.experimental.pallas import tpu as pltpu
from jax.experimental.pallas import tpu_sc as plsc

_N_SIZES = [10000, 20000, 40000]
_N_PADS = [10240, 20480, 40960]
_BN = 512
_CH = 128
_RANGE2 = 20480
_AR_EXTRA = 512


def _round_up(v, m):
    return (v + m - 1) // m * m


def _mm_body(x_ref, w_ref, o0_ref, o1_ref, *, chunks, w0w):
    y = jnp.dot(x_ref[...], w_ref[...], preferred_element_type=jnp.float32,
                precision=jax.lax.Precision.DEFAULT)
    o0_ref[...] = y[:, :w0w]
    for j in range(chunks):
        o1_ref[j] = y[:, w0w + 128 * j:w0w + 128 * (j + 1)]


def _matmul_tables(x, wcat, chunks, w0w):
    n_rows, d_pad = x.shape
    f = wcat.shape[1]
    grid = (n_rows // _BN,)
    xw0, xw1 = pl.pallas_call(
        functools.partial(_mm_body, chunks=chunks, w0w=w0w),
        grid=grid,
        in_specs=[
            pl.BlockSpec((_BN, d_pad), lambda i: (i, 0)),
            pl.BlockSpec((d_pad, f), lambda i: (0, 0)),
        ],
        out_specs=[
            pl.BlockSpec((_BN, w0w), lambda i: (i, 0)),
            pl.BlockSpec((chunks, _BN, 128), lambda i: (0, i, 0)),
        ],
        out_shape=[
            jax.ShapeDtypeStruct((n_rows, w0w), jnp.float32),
            jax.ShapeDtypeStruct((chunks, n_rows, 128), jnp.float32),
        ],
    )(x, wcat)
    return xw0, xw1.reshape(chunks * n_rows, 128)


def _pick_unroll(nch):
    return 4 if nch % 4 == 0 else (2 if nch % 2 == 0 else 1)


def _sc_seg(n_rows, ar, e_pad, chunks, ranges):
    ept = e_pad // 16
    nch = ept // _CH
    ngroups = nch // 2
    rpt = ar // 16
    erows = e_pad // _CH
    total = chunks * ranges
    pps = (total + 1) // 2
    mesh = plsc.VectorSubcoreMesh(core_axis_name="c", subcore_axis_name="s")
    scratch = ([pltpu.VMEM((2, _CH), jnp.int32) for _ in range(4)]
               + [pltpu.VMEM((_CH, 128), jnp.float32) for _ in range(2)]
               + [pltpu.VMEM_SHARED((ar, 128), jnp.float32)]
               + [pltpu.SemaphoreType.DMA for _ in range(2)])

    def body(xw1_hbm, srcoff_hbm, dstr_hbm, zeros_hbm, out_hbm, *scr):
        sb0, db0, sb1, db1 = scr[0:4]
        rows = scr[4:6]
        acc = scr[6]
        sems = scr[7:9]
        cc = lax.axis_index("c")
        ss = lax.axis_index("s")
        for p in range(pps):
            q = jnp.minimum(cc * pps + p, total - 1)
            j = q // ranges
            r = q % ranges
            pltpu.sync_copy(zeros_hbm, acc.at[pl.ds(ss * rpt, rpt)])
            plsc.subcore_barrier()
            sbase = j * erows + ss * nch
            dbase = r * erows + ss * nch
            pltpu.sync_copy(srcoff_hbm.at[pl.ds(sbase, 2)], sb0)
            pltpu.sync_copy(dstr_hbm.at[pl.ds(dbase, 2)], db0)
            pltpu.async_copy(xw1_hbm.at[sb0.at[0]], rows[0], sems[0])
            pltpu.async_copy(xw1_hbm.at[sb0.at[1]], rows[1], sems[1])

            def pair(go, carry):
                for par, sb, db, sbn, dbn in ((0, sb0, db0, sb1, db1),
                                              (1, sb1, db1, sb0, db0)):
                    g = 2 * go + par

                    @pl.when(g + 1 < ngroups)
                    def _():
                        pltpu.sync_copy(
                            srcoff_hbm.at[pl.ds(sbase + 2 * (g + 1), 2)], sbn)
                        pltpu.sync_copy(
                            dstr_hbm.at[pl.ds(dbase + 2 * (g + 1), 2)], dbn)
                    for uu in range(2):
                        k = 2 * g + uu
                        pltpu.make_async_copy(xw1_hbm.at[sb.at[uu]], rows[uu],
                                              sems[uu]).wait()
                        pltpu.sync_copy(rows[uu], acc.at[db.at[uu]], add=True)

                        @pl.when(k + 2 < nch)
                        def _():
                            pltpu.async_copy(xw1_hbm.at[sbn.at[uu]], rows[uu],
                                             sems[uu])
                return carry

            lax.fori_loop(0, ngroups // 2, pair, 0)
            plsc.subcore_barrier()
            obase = q * ar + ss * rpt
            pltpu.sync_copy(acc.at[pl.ds(ss * rpt, rpt)],
                            out_hbm.at[pl.ds(obase, rpt)])

    return pl.kernel(
        body,
        out_type=[jax.ShapeDtypeStruct((total * ar, 128), jnp.float32)],
        mesh=mesh, scratch_types=scratch)


def _ep_body(*refs, widths, act, with_res):
    o_ref = refs[-1]
    chunks = len(widths)
    xw0 = refs[0][...]
    parts = [refs[1 + j][0, 0][:, :widths[j]] for j in range(chunks)]
    t = jnp.concatenate(parts, axis=1) if chunks > 1 else parts[0]
    t = xw0 + t + refs[1 + chunks][...]
    if act:
        t = jnp.maximum(t, 0.0)
    if with_res:
        t = 0.5 * (refs[2 + chunks][...] + t)
    o_ref[...] = t


def _epilogue(xw0, aggf, widths, ranges, ar, range_size, n_rows, b, act, res):
    chunks = len(widths)
    agg = aggf.reshape(chunks, ranges, ar, 128)
    f = sum(widths)
    grid = (n_rows // _BN,)
    bpr = range_size // _BN

    def mk_map(j):
        return lambda i: (j, (i // bpr) % ranges, i % bpr, 0)

    in_specs = [pl.BlockSpec((_BN, f), lambda i: (i, 0))]
    in_specs += [pl.BlockSpec((1, 1, _BN, 128), mk_map(j))
                 for j in range(chunks)]
    in_specs.append(pl.BlockSpec((1, f), lambda i: (0, 0)))
    args = [xw0] + [agg] * chunks + [b]
    if res is not None:
        in_specs.append(pl.BlockSpec((_BN, f), lambda i: (i, 0)))
        args.append(res)
    return pl.pallas_call(
        functools.partial(_ep_body, widths=tuple(widths), act=act,
                          with_res=res is not None),
        grid=grid,
        in_specs=in_specs,
        out_specs=pl.BlockSpec((_BN, f), lambda i: (i, 0)),
        out_shape=jax.ShapeDtypeStruct((n_rows, f), jnp.float32),
    )(*args)


def _graph_conv(x_pad, p, ed, act=True, res=None):
    n_rows, d_pad = x_pad.shape
    e_pad = ed["e_pad"]
    ranges = ed["ranges"]
    range_size = ed["range_size"]
    ar = range_size + _AR_EXTRA
    dout = p["W0"].shape[1]
    if dout == 3:
        chunks, widths, w0w = 1, [16], 16
    elif dout == 96:
        chunks, widths, w0w = 1, [96], 96
    else:
        chunks, widths, w0w = 2, [128, 64], 192
    din = p["W0"].shape[0]
    w0 = jnp.pad(p["W0"], ((0, d_pad - din), (0, w0w - dout)))
    w1 = jnp.pad(p["W1"], ((0, d_pad - din), (0, 128 * chunks - dout)))
    wcat = jnp.concatenate([w0, w1], axis=1)
    b = jnp.pad(p["b"], (0, w0w - dout))[None, :]
    xw0, xw1 = _matmul_tables(x_pad, wcat, chunks, w0w)
    srcoff = ed["src"] if chunks == 1 else ed["srcoff2"]
    zeros = jnp.zeros((ar // 16, 128), jnp.float32)
    (aggf,) = _sc_seg(n_rows, ar, e_pad, chunks, ranges)(
        xw1, srcoff, ed["dstr"], zeros)
    return _epilogue(xw0, aggf, widths, ranges, ar, range_size, n_rows,
                     b, act, res)


def _conv(x, w, b, stride=1):
    y = jax.lax.conv_general_dilated(x, w, (stride, stride), "SAME",
                                     dimension_numbers=("NHWC", "HWIO", "NHWC"))
    return jax.nn.relu(y + b)


def _cnn18(img, cnn):
    x = img[None]
    feats = []
    for i in range(6):
        p = cnn[i]
        x = _conv(x, p["c1W"], p["c1b"])
        x = _conv(x, p["c2W"], p["c2b"])
        if i >= 2:
            feats.append(x[0])
        x = _conv(x, p["sW"], p["sb"], 2)
    return feats


def _bilinear(feat, u, v):
    s = feat.shape[0]
    u0 = jnp.clip(jnp.floor(u).astype(jnp.int32), 0, s - 1)
    v0 = jnp.clip(jnp.floor(v).astype(jnp.int32), 0, s - 1)
    u1 = jnp.clip(u0 + 1, 0, s - 1)
    v1 = jnp.clip(v0 + 1, 0, s - 1)
    du = (u - u0.astype(u.dtype))[:, None]
    dv = (v - v0.astype(v.dtype))[:, None]
    f00 = feat[v0, u0]
    f01 = feat[v0, u1]
    f10 = feat[v1, u0]
    f11 = feat[v1, u1]
    return (f00 * (1 - du) * (1 - dv) + f01 * du * (1 - dv)
            + f10 * (1 - du) * dv + f11 * du * dv)


def _projection(x, img_feats):
    xc, yc = x[:, 0], x[:, 1]
    parts = [x]
    for feat in img_feats:
        s = feat.shape[0]
        u = (jnp.tanh(xc) * 0.5 + 0.5) * (s - 1)
        v = (jnp.tanh(yc) * 0.5 + 0.5) * (s - 1)
        parts.append(_bilinear(feat, u, v))
    return jnp.concatenate(parts, axis=1)


def _unpool(x, idx):
    new = 0.5 * (x[idx[:, 0]] + x[idx[:, 1]])
    return jnp.concatenate([x, new], axis=0)


def _prep_edges(ei, n, n_rows, ranges, range_size):
    e = ei.shape[1]
    e_pad = _round_up(e, 8192)
    src = jnp.concatenate([ei[0], jnp.zeros((e_pad - e,), jnp.int32)])
    dst = jnp.concatenate([ei[1], jnp.full((e_pad - e,), n, jnp.int32)])
    srcoff2 = jnp.concatenate([src, src + n_rows])
    spread = range_size + (jnp.arange(e_pad, dtype=jnp.int32) % _AR_EXTRA)
    drs = []
    for r in range(ranges):
        lo = r * range_size
        inr = (dst >= lo) & (dst < lo + range_size)
        drs.append(jnp.where(inr, dst - lo, spread))
    return {"e_pad": e_pad, "src": src.reshape(-1, _CH),
            "srcoff2": srcoff2.reshape(-1, _CH),
            "dstr": jnp.concatenate(drs).reshape(-1, _CH),
            "ranges": ranges, "range_size": range_size}


_STAGE_RANGES = [(1, 10240), (2, 10240), (4, 10240)]


def kernel(img_input, features, edge_index0, edge_index1, edge_index2,
           pool_idx0, pool_idx1, params):
    eis = [edge_index0, edge_index1, edge_index2]
    pis = [pool_idx0, pool_idx1]
    img_feats = _cnn18(img_input, params["cnn"])
    x = features
    outputs, outputs_unpool = [], []
    x_conv = None
    for i in range(3):
        n = _N_SIZES[i]
        n_rows = _N_PADS[i] + _AR_EXTRA
        ranges, range_size = _STAGE_RANGES[i]
        ed = _prep_edges(eis[i], n, n_rows, ranges, range_size)
        x_proj = _projection(x, img_feats)
        if i > 0:
            outputs_unpool.append(_unpool(x, pis[i - 1]))
            x_proj = jnp.concatenate([x_proj, x_conv], axis=1)
            x_proj = _unpool(x_proj, pis[i - 1])
        d = x_proj.shape[1]
        d_pad = _round_up(d, 128)
        xp = jnp.pad(x_proj, ((0, n_rows - n), (0, d_pad - d)))
        st = params["gcn"][i]
        h = _graph_conv(xp, st["gc_in"], ed, act=True)
        for rb in st["res"]:
            h1 = _graph_conv(h, rb["gc1"], ed, act=True)
            h = _graph_conv(h1, rb["gc2"], ed, act=True, res=h)
        x_conv = h[:n]
        if i == 2:
            y = _graph_conv(h, st["final"][0], ed, act=True)
            yp = jnp.pad(y, ((0, 0), (0, 128 - y.shape[1])))
            xo = _graph_conv(yp, st["final"][1], ed, act=False)
        else:
            xo = _graph_conv(h, st["final"][0], ed, act=False)
        x = xo[:n, :3]
        outputs.append(x)
    return tuple(outputs) + tuple(outputs_unpool)


# speedup vs baseline: 1.7146x; 1.0944x over previous
import functools

import jax
import jax.numpy as jnp
from jax import lax
from jax.experimental import pallas as pl
from jax.experimental.pallas import tpu as pltpu
from jax.experimental.pallas import tpu_sc as plsc

_N_SIZES = [10000, 20000, 40000]
_N_PADS = [10240, 20480, 40960]
_BN = 512
_CH = 64
_RANGE2 = 20480
_AR_EXTRA = 256


def _round_up(v, m):
    return (v + m - 1) // m * m


def _mm_body(x_ref, w_ref, o0_ref, o1_ref, *, chunks, w0w):
    y = jnp.dot(x_ref[...], w_ref[...], preferred_element_type=jnp.float32,
                precision=jax.lax.Precision.DEFAULT)
    o0_ref[...] = y[:, :w0w]
    for j in range(chunks):
        o1_ref[j] = y[:, w0w + 128 * j:w0w + 128 * (j + 1)]


def _matmul_tables(x, wcat, chunks, w0w):
    n_rows, d_pad = x.shape
    f = wcat.shape[1]
    grid = (n_rows // _BN,)
    xw0, xw1 = pl.pallas_call(
        functools.partial(_mm_body, chunks=chunks, w0w=w0w),
        grid=grid,
        in_specs=[
            pl.BlockSpec((_BN, d_pad), lambda i: (i, 0)),
            pl.BlockSpec((d_pad, f), lambda i: (0, 0)),
        ],
        out_specs=[
            pl.BlockSpec((_BN, w0w), lambda i: (i, 0)),
            pl.BlockSpec((chunks, _BN, 128), lambda i: (0, i, 0)),
        ],
        out_shape=[
            jax.ShapeDtypeStruct((n_rows, w0w), jnp.float32),
            jax.ShapeDtypeStruct((chunks, n_rows, 128), jnp.float32),
        ],
    )(x, wcat)
    return xw0, xw1.reshape(chunks * n_rows, 128)


def _pick_unroll(nch):
    return 4 if nch % 4 == 0 else (2 if nch % 2 == 0 else 1)


def _sc_seg(n_rows, ar, e_pad, chunks, ranges):
    ept = e_pad // 16
    nch = ept // _CH
    ngroups = nch // 2
    rpt = ar // 16
    erows = e_pad // _CH
    total = chunks * ranges
    pps = (total + 1) // 2
    mesh = plsc.VectorSubcoreMesh(core_axis_name="c", subcore_axis_name="s")
    scratch = ([pltpu.VMEM((2, _CH), jnp.int32) for _ in range(4)]
               + [pltpu.VMEM((_CH, 128), jnp.float32) for _ in range(2)]
               + [pltpu.VMEM_SHARED((ar, 128), jnp.float32)]
               + [pltpu.SemaphoreType.DMA for _ in range(2)])

    def body(xw1_hbm, srcoff_hbm, dstr_hbm, zeros_hbm, out_hbm, *scr):
        sb0, db0, sb1, db1 = scr[0:4]
        rows = scr[4:6]
        acc = scr[6]
        sems = scr[7:9]
        cc = lax.axis_index("c")
        ss = lax.axis_index("s")
        for p in range(pps):
            q = jnp.minimum(cc * pps + p, total - 1)
            j = q // ranges
            r = q % ranges
            pltpu.sync_copy(zeros_hbm, acc.at[pl.ds(ss * rpt, rpt)])
            plsc.subcore_barrier()
            sbase = j * erows + ss * nch
            dbase = r * erows + ss * nch
            pltpu.sync_copy(srcoff_hbm.at[pl.ds(sbase, 2)], sb0)
            pltpu.sync_copy(dstr_hbm.at[pl.ds(dbase, 2)], db0)
            pltpu.async_copy(xw1_hbm.at[sb0.at[0]], rows[0], sems[0])
            pltpu.async_copy(xw1_hbm.at[sb0.at[1]], rows[1], sems[1])

            def pair(go, carry):
                for par, sb, db, sbn, dbn in ((0, sb0, db0, sb1, db1),
                                              (1, sb1, db1, sb0, db0)):
                    g = 2 * go + par

                    @pl.when(g + 1 < ngroups)
                    def _():
                        pltpu.sync_copy(
                            srcoff_hbm.at[pl.ds(sbase + 2 * (g + 1), 2)], sbn)
                        pltpu.sync_copy(
                            dstr_hbm.at[pl.ds(dbase + 2 * (g + 1), 2)], dbn)
                    for uu in range(2):
                        k = 2 * g + uu
                        pltpu.make_async_copy(xw1_hbm.at[sb.at[uu]], rows[uu],
                                              sems[uu]).wait()
                        pltpu.sync_copy(rows[uu], acc.at[db.at[uu]], add=True)

                        @pl.when(k + 2 < nch)
                        def _():
                            pltpu.async_copy(xw1_hbm.at[sbn.at[uu]], rows[uu],
                                             sems[uu])
                return carry

            lax.fori_loop(0, ngroups // 2, pair, 0)
            plsc.subcore_barrier()
            obase = q * ar + ss * rpt
            pltpu.sync_copy(acc.at[pl.ds(ss * rpt, rpt)],
                            out_hbm.at[pl.ds(obase, rpt)])

    return pl.kernel(
        body,
        out_type=[jax.ShapeDtypeStruct((total * ar, 128), jnp.float32)],
        mesh=mesh, scratch_types=scratch)


def _ep_body(*refs, widths, act, with_res):
    o_ref = refs[-1]
    chunks = len(widths)
    xw0 = refs[0][...]
    parts = [refs[1 + j][0, 0][:, :widths[j]] for j in range(chunks)]
    t = jnp.concatenate(parts, axis=1) if chunks > 1 else parts[0]
    t = xw0 + t + refs[1 + chunks][...]
    if act:
        t = jnp.maximum(t, 0.0)
    if with_res:
        t = 0.5 * (refs[2 + chunks][...] + t)
    o_ref[...] = t


def _epilogue(xw0, aggf, widths, ranges, ar, range_size, n_rows, b, act, res):
    chunks = len(widths)
    agg = aggf.reshape(chunks, ranges, ar, 128)
    f = sum(widths)
    grid = (n_rows // _BN,)
    bpr = range_size // _BN

    def mk_map(j):
        return lambda i: (j, (i // bpr) % ranges, i % bpr, 0)

    in_specs = [pl.BlockSpec((_BN, f), lambda i: (i, 0))]
    in_specs += [pl.BlockSpec((1, 1, _BN, 128), mk_map(j))
                 for j in range(chunks)]
    in_specs.append(pl.BlockSpec((1, f), lambda i: (0, 0)))
    args = [xw0] + [agg] * chunks + [b]
    if res is not None:
        in_specs.append(pl.BlockSpec((_BN, f), lambda i: (i, 0)))
        args.append(res)
    return pl.pallas_call(
        functools.partial(_ep_body, widths=tuple(widths), act=act,
                          with_res=res is not None),
        grid=grid,
        in_specs=in_specs,
        out_specs=pl.BlockSpec((_BN, f), lambda i: (i, 0)),
        out_shape=jax.ShapeDtypeStruct((n_rows, f), jnp.float32),
    )(*args)


def _graph_conv(x_pad, p, ed, act=True, res=None):
    n_rows, d_pad = x_pad.shape
    e_pad = ed["e_pad"]
    ranges = ed["ranges"]
    range_size = ed["range_size"]
    ar = range_size + _AR_EXTRA
    dout = p["W0"].shape[1]
    if dout == 3:
        chunks, widths, w0w = 1, [16], 16
    elif dout == 96:
        chunks, widths, w0w = 1, [96], 96
    else:
        chunks, widths, w0w = 2, [128, 64], 192
    din = p["W0"].shape[0]
    w0 = jnp.pad(p["W0"], ((0, d_pad - din), (0, w0w - dout)))
    w1 = jnp.pad(p["W1"], ((0, d_pad - din), (0, 128 * chunks - dout)))
    wcat = jnp.concatenate([w0, w1], axis=1)
    b = jnp.pad(p["b"], (0, w0w - dout))[None, :]
    xw0, xw1 = _matmul_tables(x_pad, wcat, chunks, w0w)
    srcoff = ed["src"] if chunks == 1 else ed["srcoff2"]
    zeros = jnp.zeros((ar // 16, 128), jnp.float32)
    (aggf,) = _sc_seg(n_rows, ar, e_pad, chunks, ranges)(
        xw1, srcoff, ed["dstr"], zeros)
    return _epilogue(xw0, aggf, widths, ranges, ar, range_size, n_rows,
                     b, act, res)


def _conv(x, w, b, stride=1):
    y = jax.lax.conv_general_dilated(x, w, (stride, stride), "SAME",
                                     dimension_numbers=("NHWC", "HWIO", "NHWC"))
    return jax.nn.relu(y + b)


def _cnn18(img, cnn):
    x = img[None]
    feats = []
    for i in range(6):
        p = cnn[i]
        x = _conv(x, p["c1W"], p["c1b"])
        x = _conv(x, p["c2W"], p["c2b"])
        if i >= 2:
            feats.append(x[0])
        x = _conv(x, p["sW"], p["sb"], 2)
    return feats


def _bilinear(feat, u, v):
    s = feat.shape[0]
    u0 = jnp.clip(jnp.floor(u).astype(jnp.int32), 0, s - 1)
    v0 = jnp.clip(jnp.floor(v).astype(jnp.int32), 0, s - 1)
    u1 = jnp.clip(u0 + 1, 0, s - 1)
    v1 = jnp.clip(v0 + 1, 0, s - 1)
    du = (u - u0.astype(u.dtype))[:, None]
    dv = (v - v0.astype(v.dtype))[:, None]
    f00 = feat[v0, u0]
    f01 = feat[v0, u1]
    f10 = feat[v1, u0]
    f11 = feat[v1, u1]
    return (f00 * (1 - du) * (1 - dv) + f01 * du * (1 - dv)
            + f10 * (1 - du) * dv + f11 * du * dv)


def _projection(x, img_feats):
    xc, yc = x[:, 0], x[:, 1]
    parts = [x]
    for feat in img_feats:
        s = feat.shape[0]
        u = (jnp.tanh(xc) * 0.5 + 0.5) * (s - 1)
        v = (jnp.tanh(yc) * 0.5 + 0.5) * (s - 1)
        parts.append(_bilinear(feat, u, v))
    return jnp.concatenate(parts, axis=1)


def _unpool(x, idx):
    new = 0.5 * (x[idx[:, 0]] + x[idx[:, 1]])
    return jnp.concatenate([x, new], axis=0)


def _prep_edges(ei, n, n_rows, ranges, range_size):
    e = ei.shape[1]
    e_pad = _round_up(e, 8192)
    src = jnp.concatenate([ei[0], jnp.zeros((e_pad - e,), jnp.int32)])
    dst = jnp.concatenate([ei[1], jnp.full((e_pad - e,), n, jnp.int32)])
    srcoff2 = jnp.concatenate([src, src + n_rows])
    spread = range_size + (jnp.arange(e_pad, dtype=jnp.int32) % _AR_EXTRA)
    drs = []
    for r in range(ranges):
        lo = r * range_size
        inr = (dst >= lo) & (dst < lo + range_size)
        drs.append(jnp.where(inr, dst - lo, spread))
    return {"e_pad": e_pad, "src": src.reshape(-1, _CH),
            "srcoff2": srcoff2.reshape(-1, _CH),
            "dstr": jnp.concatenate(drs).reshape(-1, _CH),
            "ranges": ranges, "range_size": range_size}


_STAGE_RANGES = [(1, 10240), (2, 10240), (3, 13824)]


def kernel(img_input, features, edge_index0, edge_index1, edge_index2,
           pool_idx0, pool_idx1, params):
    eis = [edge_index0, edge_index1, edge_index2]
    pis = [pool_idx0, pool_idx1]
    img_feats = _cnn18(img_input, params["cnn"])
    x = features
    outputs, outputs_unpool = [], []
    x_conv = None
    for i in range(3):
        n = _N_SIZES[i]
        n_rows = _N_PADS[i] + _AR_EXTRA
        ranges, range_size = _STAGE_RANGES[i]
        ed = _prep_edges(eis[i], n, n_rows, ranges, range_size)
        x_proj = _projection(x, img_feats)
        if i > 0:
            outputs_unpool.append(_unpool(x, pis[i - 1]))
            x_proj = jnp.concatenate([x_proj, x_conv], axis=1)
            x_proj = _unpool(x_proj, pis[i - 1])
        d = x_proj.shape[1]
        d_pad = _round_up(d, 128)
        xp = jnp.pad(x_proj, ((0, n_rows - n), (0, d_pad - d)))
        st = params["gcn"][i]
        h = _graph_conv(xp, st["gc_in"], ed, act=True)
        for rb in st["res"]:
            h1 = _graph_conv(h, rb["gc1"], ed, act=True)
            h = _graph_conv(h1, rb["gc2"], ed, act=True, res=h)
        x_conv = h[:n]
        if i == 2:
            y = _graph_conv(h, st["final"][0], ed, act=True)
            yp = jnp.pad(y, ((0, 0), (0, 128 - y.shape[1])))
            xo = _graph_conv(yp, st["final"][1], ed, act=False)
        else:
            xo = _graph_conv(h, st["final"][0], ed, act=False)
        x = xo[:n, :3]
        outputs.append(x)
    return tuple(outputs) + tuple(outputs_unpool)
